# Initial kernel scaffold; baseline (speedup 1.0000x reference)
import jax
import jax.numpy as jnp
from jax.experimental import pallas as pl


def kernel(pred_feat, pred_occ, radar_feat, lidar_feat, radar_indices, lidar_indices):
    raise NotImplementedError("write your pallas kernel here")


# speedup vs baseline: 925.9819x; 925.9819x over previous
import jax
import jax.numpy as jnp
from jax import lax
from jax.experimental import pallas as pl
from jax.experimental.pallas import tpu as pltpu
from jax.experimental.pallas import tpu_sc as plsc

Z, Y, X = 41, 1440, 1440
BB = 4
YX = Y * X
HASH_N = BB * Z * Y * X
TABLE_N = HASH_N + YX
NR = 100000
NL = 400000
NT = 16
NRP = 100352
NLP = 401408
R_ROWS = NRP // (NT * 128)
L_ROWS = NLP // (NT * 128)
K = 27
BIGKEY = 1024
MIX = -1640531527
VCAP = 2048
_KEYC = []
for _k in range(K):
    _dz, _dy, _dx = _k // 9 - 1, (_k // 3) % 3 - 1, _k % 3 - 1
    _KEYC.append((abs(_dz) + abs(_dy) + abs(_dx)) * 32 + _k)


def _sc_body(lb, lz, ly, lx, rb, rz, ry, rx, feat0,
             m_out, gdz_out, gdy_out, gdx_out, gf0_out, hl_out, table,
             hv_all, cb, cz, cy, cx, ival, tvf,
             vlh, vlv, vgh, vgv, tvl,
             prb, prz, pry, prx, cand, tv, cidx, hchk,
             outm, odz, ody, odx, of0, fidx, fv, nqv, nqall,
             shared, sem):
    wid = lax.axis_index("s").astype(jnp.int32)
    c512 = jnp.int32(512)
    c128 = jnp.int32(128)
    c4 = jnp.int32(4)
    mix = jnp.int32(MIX)
    lbase = wid * jnp.int32(L_ROWS * 128)
    rbase = wid * jnp.int32(R_ROWS * 128)
    it16 = lax.iota(jnp.int32, 16)

    @pl.loop(jnp.int32(0), jnp.int32(L_ROWS // 4))
    def _build(blk):
        blk = blk.astype(jnp.int32)
        g = lbase + blk * c512
        cps = [pltpu.async_copy(lb.at[pl.ds(g, 512)], cb, sem),
               pltpu.async_copy(lz.at[pl.ds(g, 512)], cz, sem),
               pltpu.async_copy(ly.at[pl.ds(g, 512)], cy, sem),
               pltpu.async_copy(lx.at[pl.ds(g, 512)], cx, sem)]
        for cp in cps:
            cp.wait()
        for r in range(4):
            for c in range(8):
                s = r * 128 + c * 16
                h = ((cb[pl.ds(s, 16)] * Z + cz[pl.ds(s, 16)]) * Y
                     + cy[pl.ds(s, 16)]) * X + cx[pl.ds(s, 16)]
                hv_all[blk * c4 + jnp.int32(r), pl.ds(c * 16, 16)] = h
                ival[jnp.int32(r), pl.ds(c * 16, 16)] = (
                    (g + jnp.int32(s) + it16) ^ (h * mix))
        cps = []
        for r in range(4):
            row = blk * c4 + jnp.int32(r)
            cps.append(pltpu.async_copy(
                ival.at[jnp.int32(r)], table.at[hv_all.at[row]], sem))
            cps.append(pltpu.async_copy(
                hv_all.at[row], hl_out.at[pl.ds(g + jnp.int32(r * 128), 128)],
                sem))
        for cp in cps:
            cp.wait()

    plsc.subcore_barrier()

    redirect = jnp.int32(TABLE_N - 1024) + wid * jnp.int32(64)

    for j in range(16):
        for c in range(8):
            off = j * 128 + c * 16
            vlh[pl.ds(off, 16)] = redirect + it16 + jnp.int32(((j * 8 + c) & 3) * 16)
            vlv[pl.ds(off, 16)] = jnp.int32(1 << 30) + it16

    def _collect(blk, cnt):
        blk = blk.astype(jnp.int32)
        cps = []
        for r in range(4):
            cps.append(pltpu.async_copy(
                table.at[hv_all.at[blk * c4 + jnp.int32(r)]],
                tvf.at[jnp.int32(r)], sem))
        for cp in cps:
            cp.wait()
        for r in range(4):
            for c in range(8):
                sl = pl.ds(c * 16, 16)
                h = hv_all[blk * c4 + jnp.int32(r), sl]
                dec = tvf[jnp.int32(r), sl] ^ (h * mix)
                i16 = (lbase + blk * c512 + jnp.int32(r * 128 + c * 16)
                       + it16)
                q = i16 < dec
                off = jnp.minimum(cnt, jnp.int32(VCAP - 16))
                plsc.store_compressed(vlh.at[pl.ds(off, 16)], h, mask=q)
                plsc.store_compressed(vlv.at[pl.ds(off, 16)], i16, mask=q)
                pc = jnp.sum(jnp.where(q, jnp.int32(1), jnp.int32(0)),
                             axis=0, dtype=jnp.int32)
                cnt = jnp.minimum(off + pc, jnp.int32(VCAP - 16))
        return cnt

    lax.fori_loop(jnp.int32(0), jnp.int32(L_ROWS // 4), _collect,
                  jnp.int32(0))

    def _round(carry):
        cps = []
        for j in range(16):
            cps.append(pltpu.async_copy(
                table.at[vlh.at[pl.ds(j * 128, 128)]], tvl.at[jnp.int32(j)],
                sem))
        for cp in cps:
            cp.wait()
        nqvec = jnp.zeros((16,), jnp.int32)
        for j in range(16):
            for c in range(8):
                sl = pl.ds(c * 16, 16)
                lsl = pl.ds(j * 128 + c * 16, 16)
                h = vlh[lsl]
                i = vlv[lsl]
                dec = tvl[jnp.int32(j), sl] ^ (h * mix)
                q = i < dec
                vgh[jnp.int32(j), sl] = jnp.where(
                    q, h, redirect + it16 + jnp.int32(((j * 8 + c) & 3) * 16))
                vgv[jnp.int32(j), sl] = jnp.where(q, i ^ (h * mix), 0)
                nqvec = nqvec + jnp.where(q, jnp.int32(1), jnp.int32(0))
        cps = []
        for j in range(16):
            cps.append(pltpu.async_copy(
                vgv.at[jnp.int32(j)], table.at[vgh.at[jnp.int32(j)]], sem))
        for cp in cps:
            cp.wait()
        nqv[pl.ds(0, 16)] = nqvec
        pltpu.sync_copy(nqv, shared.at[wid])
        plsc.subcore_barrier()
        pltpu.sync_copy(shared, nqall)
        plsc.subcore_barrier()
        tot = jnp.zeros((16,), jnp.int32)
        for t in range(NT):
            tot = tot + nqall[jnp.int32(t), pl.ds(0, 16)]
        return jnp.sum(tot, axis=0, dtype=jnp.int32)

    lax.while_loop(lambda c: c > 0, _round, jnp.int32(1))

    lowmask = jnp.int32(0x3FFFF)
    nlp = jnp.int32(NLP)

    @pl.loop(jnp.int32(0), jnp.int32(R_ROWS))
    def _probe(blk):
        blk = blk.astype(jnp.int32)
        g = rbase + blk * c128
        cps = [pltpu.async_copy(rb.at[pl.ds(g, 128)], prb, sem),
               pltpu.async_copy(rz.at[pl.ds(g, 128)], prz, sem),
               pltpu.async_copy(ry.at[pl.ds(g, 128)], pry, sem),
               pltpu.async_copy(rx.at[pl.ds(g, 128)], prx, sem)]
        for cp in cps:
            cp.wait()
        for c in range(8):
            sl = pl.ds(c * 16, 16)
            bv = prb[sl]; zv = prz[sl]; yv = pry[sl]; xv = prx[sl]
            h0 = ((bv * Z + zv) * Y + yv) * X + xv
            zero = jnp.zeros((16,), jnp.int32)
            szm = jnp.where(zv > 0, jnp.int32(-YX), 0)
            szp = jnp.where(zv < Z - 1, jnp.int32(YX), 0)
            sym = jnp.where(yv > 0, jnp.int32(-X), 0)
            syp = jnp.where(yv < Y - 1, jnp.int32(X), 0)
            sxm = jnp.where(xv > 0, jnp.int32(-1), 0)
            sxp = jnp.where(xv < X - 1, jnp.int32(1), 0)
            zt = (szm, zero, szp)
            yt = (sym, zero, syp)
            xt = (sxm, zero, sxp)
            for k in range(K):
                t = h0 + zt[k // 9] + yt[(k // 3) % 3] + xt[k % 3]
                cand[jnp.int32(k), sl] = t
        cps = [pltpu.async_copy(table.at[cand.at[jnp.int32(k)]],
                                tv.at[jnp.int32(k)], sem)
               for k in range(K)]
        for cp in cps:
            cp.wait()
        for k in range(K):
            for c in range(8):
                sl = pl.ds(c * 16, 16)
                cv = cand[jnp.int32(k), sl]
                dec = tv[jnp.int32(k), sl] ^ (cv * mix)
                tv[jnp.int32(k), sl] = dec
                cidx[jnp.int32(k), sl] = jnp.where(
                    (dec >= 0) & (dec < nlp), dec, dec & lowmask)
        cps = [pltpu.async_copy(hl_out.at[cidx.at[jnp.int32(k)]],
                                hchk.at[jnp.int32(k)], sem)
               for k in range(K)]
        for cp in cps:
            cp.wait()
        for c in range(8):
            sl = pl.ds(c * 16, 16)
            best = jnp.full((16,), BIGKEY, jnp.int32)
            for k in range(K):
                hit = hchk[jnp.int32(k), sl] == cand[jnp.int32(k), sl]
                best = jnp.minimum(
                    best, jnp.where(hit, jnp.int32(_KEYC[k]), BIGKEY))
            bk = best & 31
            mt = best < BIGKEY
            q9 = lax.shift_right_logical(bk * 57, jnp.int32(9))
            rem = bk - q9 * 9
            q3 = lax.shift_right_logical(rem * 171, jnp.int32(9))
            dzk = q9 - 1
            dyk = q3 - 1
            dxk = rem - q3 * 3 - 1
            zv = prz[sl]; yv = pry[sl]; xv = prx[sl]
            cdz = jnp.clip(zv + dzk, 0, Z - 1) - zv
            cdy = jnp.clip(yv + dyk, 0, Y - 1) - yv
            cdx = jnp.clip(xv + dxk, 0, X - 1) - xv
            row = plsc.load_gather(tv, [bk, c * 16 + it16])
            outm[sl] = jnp.where(mt, jnp.float32(1.0), jnp.float32(0.0))
            odz[sl] = cdz.astype(jnp.float32)
            ody[sl] = cdy.astype(jnp.float32)
            odx[sl] = cdx.astype(jnp.float32)
            fidx[sl] = jnp.where((row >= 0) & (row < nlp), row, row & lowmask)
        pltpu.async_copy(feat0.at[fidx], fv, sem).wait()
        for c in range(8):
            sl = pl.ds(c * 16, 16)
            of0[sl] = jnp.where(outm[sl] > 0.5, fv[sl], 0.0)
        pltpu.sync_copy(outm, m_out.at[pl.ds(g, 128)])
        pltpu.sync_copy(odz, gdz_out.at[pl.ds(g, 128)])
        pltpu.sync_copy(ody, gdy_out.at[pl.ds(g, 128)])
        pltpu.sync_copy(odx, gdx_out.at[pl.ds(g, 128)])
        pltpu.sync_copy(of0, gf0_out.at[pl.ds(g, 128)])


def _sc_match(lb, lz, ly, lx, rb, rz, ry, rx, feat0):
    mesh = plsc.VectorSubcoreMesh(
        core_axis_name="c", subcore_axis_name="s", num_cores=1)
    f = pl.kernel(
        _sc_body,
        out_type=(
            jax.ShapeDtypeStruct((NRP,), jnp.float32),
            jax.ShapeDtypeStruct((NRP,), jnp.float32),
            jax.ShapeDtypeStruct((NRP,), jnp.float32),
            jax.ShapeDtypeStruct((NRP,), jnp.float32),
            jax.ShapeDtypeStruct((NRP,), jnp.float32),
            jax.ShapeDtypeStruct((NLP,), jnp.int32),
            jax.ShapeDtypeStruct((TABLE_N,), jnp.int32),
        ),
        mesh=mesh,
        compiler_params=pltpu.CompilerParams(needs_layout_passes=False),
        scratch_types=[
            pltpu.VMEM((L_ROWS, 128), jnp.int32),
            pltpu.VMEM((512,), jnp.int32),
            pltpu.VMEM((512,), jnp.int32),
            pltpu.VMEM((512,), jnp.int32),
            pltpu.VMEM((512,), jnp.int32),
            pltpu.VMEM((4, 128), jnp.int32),
            pltpu.VMEM((4, 128), jnp.int32),
            pltpu.VMEM((VCAP,), jnp.int32),
            pltpu.VMEM((VCAP,), jnp.int32),
            pltpu.VMEM((16, 128), jnp.int32),
            pltpu.VMEM((16, 128), jnp.int32),
            pltpu.VMEM((16, 128), jnp.int32),
            pltpu.VMEM((128,), jnp.int32),
            pltpu.VMEM((128,), jnp.int32),
            pltpu.VMEM((128,), jnp.int32),
            pltpu.VMEM((128,), jnp.int32),
            pltpu.VMEM((K, 128), jnp.int32),
            pltpu.VMEM((K, 128), jnp.int32),
            pltpu.VMEM((K, 128), jnp.int32),
            pltpu.VMEM((K, 128), jnp.int32),
            pltpu.VMEM((128,), jnp.float32),
            pltpu.VMEM((128,), jnp.float32),
            pltpu.VMEM((128,), jnp.float32),
            pltpu.VMEM((128,), jnp.float32),
            pltpu.VMEM((128,), jnp.float32),
            pltpu.VMEM((128,), jnp.int32),
            pltpu.VMEM((128,), jnp.float32),
            pltpu.VMEM((16,), jnp.int32),
            pltpu.VMEM((NT, 16), jnp.int32),
            pltpu.VMEM_SHARED((NT, 16), jnp.int32),
            pltpu.SemaphoreType.DMA,
        ],
        name="synth_local_match_sc",
    )
    return f(lb, lz, ly, lx, rb, rz, ry, rx, feat0)


def _loss_body(po, p0, p1, p2, p3, m, gz, gy, gx, gf, out_ref):
    rows = po.shape[0]
    ridx = lax.broadcasted_iota(jnp.int32, (rows, 128), 0)
    cidx = lax.broadcasted_iota(jnp.int32, (rows, 128), 1)
    valid = (ridx * 128 + cidx < NR).astype(jnp.float32)
    o = po[...]
    mv = m[...] * valid
    bce = (jnp.maximum(o, 0.0) - o * mv
           + jnp.log1p(jnp.exp(-jnp.abs(o)))) * valid
    occ_loss = jnp.sum(bce) / NR
    cnt = jnp.maximum(jnp.sum(mv), 1.0)

    def smooth(d):
        ad = jnp.abs(d)
        return jnp.where(ad < 1.0, 0.5 * d * d, ad - 0.5)

    off = (smooth(p0[...] - gz[...]) + smooth(p1[...] - gy[...])
           + smooth(p2[...] - gx[...]))
    off_loss = jnp.sum(off * mv) / (cnt * 3.0)
    feat_loss = jnp.sum(jnp.abs(p3[...] - gf[...]) * mv) / cnt
    out_ref[0, 0] = 0.2 * occ_loss + off_loss + feat_loss


def _tc_loss(po, p0, p1, p2, p3, m, gz, gy, gx, gf):
    rows = NRP // 128
    f = pl.pallas_call(
        _loss_body,
        out_shape=jax.ShapeDtypeStruct((1, 1), jnp.float32),
        out_specs=pl.BlockSpec(memory_space=pltpu.SMEM),
        name="synth_local_loss_tc",
    )
    args = [a.reshape(rows, 128) for a in (po, p0, p1, p2, p3, m, gz, gy, gx, gf)]
    return f(*args)


def kernel(pred_feat, pred_occ, radar_feat, lidar_feat,
           radar_indices, lidar_indices):
    del radar_feat
    padl = NLP - NL
    padr = NRP - NR
    li = lidar_indices.astype(jnp.int32)
    ri = radar_indices.astype(jnp.int32)
    lb = jnp.concatenate([li[:, 0], jnp.full((padl,), BB, jnp.int32)])
    lz = jnp.concatenate([li[:, 1], jnp.zeros((padl,), jnp.int32)])
    ly = jnp.concatenate([li[:, 2],
                          (jnp.arange(padl, dtype=jnp.int32) % Y)])
    lx = jnp.concatenate([li[:, 3], jnp.zeros((padl,), jnp.int32)])
    zr = jnp.zeros((padr,), jnp.int32)
    rb = jnp.concatenate([ri[:, 0], zr])
    rz = jnp.concatenate([ri[:, 1], zr])
    ry = jnp.concatenate([ri[:, 2], zr])
    rx = jnp.concatenate([ri[:, 3], zr])

    feat0 = jnp.concatenate([lidar_feat[:, 0], jnp.zeros((padl,), jnp.float32)])
    m, gz, gy, gx, gf, _hl, _table = _sc_match(
        lb, lz, ly, lx, rb, rz, ry, rx, feat0)

    po = jnp.concatenate([pred_occ[:, 0], jnp.zeros((padr,), jnp.float32)])
    p0 = jnp.concatenate([pred_feat[:, 0], jnp.zeros((padr,), jnp.float32)])
    p1 = jnp.concatenate([pred_feat[:, 1], jnp.zeros((padr,), jnp.float32)])
    p2 = jnp.concatenate([pred_feat[:, 2], jnp.zeros((padr,), jnp.float32)])
    p3 = jnp.concatenate([pred_feat[:, 3], jnp.zeros((padr,), jnp.float32)])
    out = _tc_loss(po, p0, p1, p2, p3, m, gz, gy, gx, gf)
    return out.reshape(())


# speedup vs baseline: 1464.5054x; 1.5816x over previous
import jax
import jax.numpy as jnp
from jax import lax
from jax.experimental import pallas as pl
from jax.experimental.pallas import tpu as pltpu
from jax.experimental.pallas import tpu_sc as plsc

Z, Y, X = 41, 1440, 1440
BB = 4
YX = Y * X
HASH_N = BB * Z * Y * X
TABLE_N = HASH_N + YX
NR = 100000
NL = 400000
NT = 16
NRP = 100352
NLP = 401408
R_ROWS = NRP // (NT * 128)
L_ROWS = NLP // (NT * 128)
K = 27
BIGKEY = 1024
MIX = -1640531527
VCAP = 2048
_KEYC = []
for _k in range(K):
    _dz, _dy, _dx = _k // 9 - 1, (_k // 3) % 3 - 1, _k % 3 - 1
    _KEYC.append((abs(_dz) + abs(_dy) + abs(_dx)) * 32 + _k)


def _sc_body(lb, lz, ly, lx, rb, rz, ry, rx, feat0,
             m_out, gdz_out, gdy_out, gdx_out, gf0_out, hl_out, table,
             hv_all, cb, cz, cy, cx, ival, tvf,
             vlh, vlv, vgh, vgv, tvl,
             prb, prz, pry, prx, cand, tv, cidx, hchk,
             outm, odz, ody, odx, of0, fidx, fv, nqv, nqall,
             shared, sem):
    wid = lax.axis_index("s").astype(jnp.int32)
    c512 = jnp.int32(512)
    c128 = jnp.int32(128)
    c4 = jnp.int32(4)
    mix = jnp.int32(MIX)
    lbase = wid * jnp.int32(L_ROWS * 128)
    rbase = wid * jnp.int32(R_ROWS * 128)
    it16 = lax.iota(jnp.int32, 16)

    @pl.loop(jnp.int32(0), jnp.int32(L_ROWS // 4))
    def _build(blk):
        blk = blk.astype(jnp.int32)
        g = lbase + blk * c512
        cps = [pltpu.async_copy(lb.at[pl.ds(g, 512)], cb, sem),
               pltpu.async_copy(lz.at[pl.ds(g, 512)], cz, sem),
               pltpu.async_copy(ly.at[pl.ds(g, 512)], cy, sem),
               pltpu.async_copy(lx.at[pl.ds(g, 512)], cx, sem)]
        for cp in cps:
            cp.wait()
        for r in range(4):
            for c in range(8):
                s = r * 128 + c * 16
                h = ((cb[pl.ds(s, 16)] * Z + cz[pl.ds(s, 16)]) * Y
                     + cy[pl.ds(s, 16)]) * X + cx[pl.ds(s, 16)]
                hv_all[blk * c4 + jnp.int32(r), pl.ds(c * 16, 16)] = h
                ival[jnp.int32(r), pl.ds(c * 16, 16)] = (
                    (g + jnp.int32(s) + it16) ^ (h * mix))
        cps = []
        for r in range(4):
            row = blk * c4 + jnp.int32(r)
            cps.append(pltpu.async_copy(
                ival.at[jnp.int32(r)], table.at[hv_all.at[row]], sem))
            cps.append(pltpu.async_copy(
                hv_all.at[row], hl_out.at[pl.ds(g + jnp.int32(r * 128), 128)],
                sem))
        for cp in cps:
            cp.wait()

    plsc.subcore_barrier()

    redirect = jnp.int32(TABLE_N - 1024) + wid * jnp.int32(64)

    for j in range(16):
        for c in range(8):
            off = j * 128 + c * 16
            vlh[pl.ds(off, 16)] = redirect + it16 + jnp.int32(((j * 8 + c) & 3) * 16)
            vlv[pl.ds(off, 16)] = jnp.int32(1 << 30) + it16

    def _collect(blk, cnt):
        blk = blk.astype(jnp.int32)
        cps = []
        for r in range(4):
            cps.append(pltpu.async_copy(
                table.at[hv_all.at[blk * c4 + jnp.int32(r)]],
                tvf.at[jnp.int32(r)], sem))
        for cp in cps:
            cp.wait()
        for r in range(4):
            for c in range(8):
                sl = pl.ds(c * 16, 16)
                h = hv_all[blk * c4 + jnp.int32(r), sl]
                dec = tvf[jnp.int32(r), sl] ^ (h * mix)
                i16 = (lbase + blk * c512 + jnp.int32(r * 128 + c * 16)
                       + it16)
                q = i16 < dec
                off = jnp.minimum(cnt, jnp.int32(VCAP - 16))
                plsc.store_compressed(vlh.at[pl.ds(off, 16)], h, mask=q)
                plsc.store_compressed(vlv.at[pl.ds(off, 16)], i16, mask=q)
                pc = jnp.sum(jnp.where(q, jnp.int32(1), jnp.int32(0)),
                             axis=0, dtype=jnp.int32)
                cnt = jnp.minimum(off + pc, jnp.int32(VCAP - 16))
        return cnt


    def _round(carry):
        cps = []
        for j in range(16):
            cps.append(pltpu.async_copy(
                table.at[vlh.at[pl.ds(j * 128, 128)]], tvl.at[jnp.int32(j)],
                sem))
        for cp in cps:
            cp.wait()
        nqvec = jnp.zeros((16,), jnp.int32)
        for j in range(16):
            for c in range(8):
                sl = pl.ds(c * 16, 16)
                lsl = pl.ds(j * 128 + c * 16, 16)
                h = vlh[lsl]
                i = vlv[lsl]
                dec = tvl[jnp.int32(j), sl] ^ (h * mix)
                q = i < dec
                vgh[jnp.int32(j), sl] = jnp.where(
                    q, h, redirect + it16 + jnp.int32(((j * 8 + c) & 3) * 16))
                vgv[jnp.int32(j), sl] = jnp.where(q, i ^ (h * mix), 0)
                nqvec = nqvec + jnp.where(q, jnp.int32(1), jnp.int32(0))
        cps = []
        for j in range(16):
            cps.append(pltpu.async_copy(
                vgv.at[jnp.int32(j)], table.at[vgh.at[jnp.int32(j)]], sem))
        for cp in cps:
            cp.wait()
        nqv[pl.ds(0, 16)] = nqvec
        pltpu.sync_copy(nqv, shared.at[wid])
        plsc.subcore_barrier()
        pltpu.sync_copy(shared, nqall)
        plsc.subcore_barrier()
        tot = jnp.zeros((16,), jnp.int32)
        for t in range(NT):
            tot = tot + nqall[jnp.int32(t), pl.ds(0, 16)]
        return jnp.sum(tot, axis=0, dtype=jnp.int32)


    lowmask = jnp.int32(0x3FFFF)
    nlp = jnp.int32(NLP)

    @pl.loop(jnp.int32(0), jnp.int32(R_ROWS))
    def _probe(blk):
        blk = blk.astype(jnp.int32)
        g = rbase + blk * c128
        cps = [pltpu.async_copy(rb.at[pl.ds(g, 128)], prb, sem),
               pltpu.async_copy(rz.at[pl.ds(g, 128)], prz, sem),
               pltpu.async_copy(ry.at[pl.ds(g, 128)], pry, sem),
               pltpu.async_copy(rx.at[pl.ds(g, 128)], prx, sem)]
        for cp in cps:
            cp.wait()
        for c in range(8):
            sl = pl.ds(c * 16, 16)
            bv = prb[sl]; zv = prz[sl]; yv = pry[sl]; xv = prx[sl]
            h0 = ((bv * Z + zv) * Y + yv) * X + xv
            zero = jnp.zeros((16,), jnp.int32)
            szm = jnp.where(zv > 0, jnp.int32(-YX), 0)
            szp = jnp.where(zv < Z - 1, jnp.int32(YX), 0)
            sym = jnp.where(yv > 0, jnp.int32(-X), 0)
            syp = jnp.where(yv < Y - 1, jnp.int32(X), 0)
            sxm = jnp.where(xv > 0, jnp.int32(-1), 0)
            sxp = jnp.where(xv < X - 1, jnp.int32(1), 0)
            zt = (szm, zero, szp)
            yt = (sym, zero, syp)
            xt = (sxm, zero, sxp)
            for k in range(K):
                t = h0 + zt[k // 9] + yt[(k // 3) % 3] + xt[k % 3]
                cand[jnp.int32(k), sl] = t
        cps = [pltpu.async_copy(table.at[cand.at[jnp.int32(k)]],
                                tv.at[jnp.int32(k)], sem)
               for k in range(K)]
        for cp in cps:
            cp.wait()
        for k in range(K):
            for c in range(8):
                sl = pl.ds(c * 16, 16)
                cv = cand[jnp.int32(k), sl]
                dec = tv[jnp.int32(k), sl] ^ (cv * mix)
                tv[jnp.int32(k), sl] = dec
                cidx[jnp.int32(k), sl] = jnp.where(
                    (dec >= 0) & (dec < nlp), dec, dec & lowmask)
        cps = [pltpu.async_copy(hl_out.at[cidx.at[jnp.int32(k)]],
                                hchk.at[jnp.int32(k)], sem)
               for k in range(K)]
        for cp in cps:
            cp.wait()
        for c in range(8):
            sl = pl.ds(c * 16, 16)
            best = jnp.full((16,), BIGKEY, jnp.int32)
            for k in range(K):
                hit = hchk[jnp.int32(k), sl] == cand[jnp.int32(k), sl]
                best = jnp.minimum(
                    best, jnp.where(hit, jnp.int32(_KEYC[k]), BIGKEY))
            bk = best & 31
            mt = best < BIGKEY
            q9 = lax.shift_right_logical(bk * 57, jnp.int32(9))
            rem = bk - q9 * 9
            q3 = lax.shift_right_logical(rem * 171, jnp.int32(9))
            dzk = q9 - 1
            dyk = q3 - 1
            dxk = rem - q3 * 3 - 1
            zv = prz[sl]; yv = pry[sl]; xv = prx[sl]
            cdz = jnp.clip(zv + dzk, 0, Z - 1) - zv
            cdy = jnp.clip(yv + dyk, 0, Y - 1) - yv
            cdx = jnp.clip(xv + dxk, 0, X - 1) - xv
            row = plsc.load_gather(tv, [bk, c * 16 + it16])
            outm[sl] = jnp.where(mt, jnp.float32(1.0), jnp.float32(0.0))
            odz[sl] = cdz.astype(jnp.float32)
            ody[sl] = cdy.astype(jnp.float32)
            odx[sl] = cdx.astype(jnp.float32)
            fidx[sl] = jnp.where((row >= 0) & (row < nlp), row, row & lowmask)
        pltpu.async_copy(feat0.at[fidx], fv, sem).wait()
        for c in range(8):
            sl = pl.ds(c * 16, 16)
            of0[sl] = jnp.where(outm[sl] > 0.5, fv[sl], 0.0)
        pltpu.sync_copy(outm, m_out.at[pl.ds(g, 128)])
        pltpu.sync_copy(odz, gdz_out.at[pl.ds(g, 128)])
        pltpu.sync_copy(ody, gdy_out.at[pl.ds(g, 128)])
        pltpu.sync_copy(odx, gdx_out.at[pl.ds(g, 128)])
        pltpu.sync_copy(of0, gf0_out.at[pl.ds(g, 128)])


def _sc_match(lb, lz, ly, lx, rb, rz, ry, rx, feat0):
    mesh = plsc.VectorSubcoreMesh(
        core_axis_name="c", subcore_axis_name="s", num_cores=1)
    f = pl.kernel(
        _sc_body,
        out_type=(
            jax.ShapeDtypeStruct((NRP,), jnp.float32),
            jax.ShapeDtypeStruct((NRP,), jnp.float32),
            jax.ShapeDtypeStruct((NRP,), jnp.float32),
            jax.ShapeDtypeStruct((NRP,), jnp.float32),
            jax.ShapeDtypeStruct((NRP,), jnp.float32),
            jax.ShapeDtypeStruct((NLP,), jnp.int32),
            jax.ShapeDtypeStruct((TABLE_N,), jnp.int32),
        ),
        mesh=mesh,
        compiler_params=pltpu.CompilerParams(needs_layout_passes=False),
        scratch_types=[
            pltpu.VMEM((L_ROWS, 128), jnp.int32),
            pltpu.VMEM((512,), jnp.int32),
            pltpu.VMEM((512,), jnp.int32),
            pltpu.VMEM((512,), jnp.int32),
            pltpu.VMEM((512,), jnp.int32),
            pltpu.VMEM((4, 128), jnp.int32),
            pltpu.VMEM((4, 128), jnp.int32),
            pltpu.VMEM((VCAP,), jnp.int32),
            pltpu.VMEM((VCAP,), jnp.int32),
            pltpu.VMEM((16, 128), jnp.int32),
            pltpu.VMEM((16, 128), jnp.int32),
            pltpu.VMEM((16, 128), jnp.int32),
            pltpu.VMEM((128,), jnp.int32),
            pltpu.VMEM((128,), jnp.int32),
            pltpu.VMEM((128,), jnp.int32),
            pltpu.VMEM((128,), jnp.int32),
            pltpu.VMEM((K, 128), jnp.int32),
            pltpu.VMEM((K, 128), jnp.int32),
            pltpu.VMEM((K, 128), jnp.int32),
            pltpu.VMEM((K, 128), jnp.int32),
            pltpu.VMEM((128,), jnp.float32),
            pltpu.VMEM((128,), jnp.float32),
            pltpu.VMEM((128,), jnp.float32),
            pltpu.VMEM((128,), jnp.float32),
            pltpu.VMEM((128,), jnp.float32),
            pltpu.VMEM((128,), jnp.int32),
            pltpu.VMEM((128,), jnp.float32),
            pltpu.VMEM((16,), jnp.int32),
            pltpu.VMEM((NT, 16), jnp.int32),
            pltpu.VMEM_SHARED((NT, 16), jnp.int32),
            pltpu.SemaphoreType.DMA,
        ],
        name="synth_local_match_sc",
    )
    return f(lb, lz, ly, lx, rb, rz, ry, rx, feat0)


def _loss_body(po, p0, p1, p2, p3, m, gz, gy, gx, gf, out_ref):
    rows = po.shape[0]
    ridx = lax.broadcasted_iota(jnp.int32, (rows, 128), 0)
    cidx = lax.broadcasted_iota(jnp.int32, (rows, 128), 1)
    valid = (ridx * 128 + cidx < NR).astype(jnp.float32)
    o = po[...]
    mv = m[...] * valid
    bce = (jnp.maximum(o, 0.0) - o * mv
           + jnp.log1p(jnp.exp(-jnp.abs(o)))) * valid
    occ_loss = jnp.sum(bce) / NR
    cnt = jnp.maximum(jnp.sum(mv), 1.0)

    def smooth(d):
        ad = jnp.abs(d)
        return jnp.where(ad < 1.0, 0.5 * d * d, ad - 0.5)

    off = (smooth(p0[...] - gz[...]) + smooth(p1[...] - gy[...])
           + smooth(p2[...] - gx[...]))
    off_loss = jnp.sum(off * mv) / (cnt * 3.0)
    feat_loss = jnp.sum(jnp.abs(p3[...] - gf[...]) * mv) / cnt
    out_ref[0, 0] = 0.2 * occ_loss + off_loss + feat_loss


def _tc_loss(po, p0, p1, p2, p3, m, gz, gy, gx, gf):
    rows = NRP // 128
    f = pl.pallas_call(
        _loss_body,
        out_shape=jax.ShapeDtypeStruct((1, 1), jnp.float32),
        out_specs=pl.BlockSpec(memory_space=pltpu.SMEM),
        name="synth_local_loss_tc",
    )
    args = [a.reshape(rows, 128) for a in (po, p0, p1, p2, p3, m, gz, gy, gx, gf)]
    return f(*args)


def kernel(pred_feat, pred_occ, radar_feat, lidar_feat,
           radar_indices, lidar_indices):
    del radar_feat
    padl = NLP - NL
    padr = NRP - NR
    li = lidar_indices.astype(jnp.int32)
    ri = radar_indices.astype(jnp.int32)
    lb = jnp.concatenate([li[:, 0], jnp.full((padl,), BB, jnp.int32)])
    lz = jnp.concatenate([li[:, 1], jnp.zeros((padl,), jnp.int32)])
    ly = jnp.concatenate([li[:, 2],
                          (jnp.arange(padl, dtype=jnp.int32) % Y)])
    lx = jnp.concatenate([li[:, 3], jnp.zeros((padl,), jnp.int32)])
    zr = jnp.zeros((padr,), jnp.int32)
    rb = jnp.concatenate([ri[:, 0], zr])
    rz = jnp.concatenate([ri[:, 1], zr])
    ry = jnp.concatenate([ri[:, 2], zr])
    rx = jnp.concatenate([ri[:, 3], zr])

    feat0 = jnp.concatenate([lidar_feat[:, 0], jnp.zeros((padl,), jnp.float32)])
    m, gz, gy, gx, gf, _hl, _table = _sc_match(
        lb, lz, ly, lx, rb, rz, ry, rx, feat0)

    po = jnp.concatenate([pred_occ[:, 0], jnp.zeros((padr,), jnp.float32)])
    p0 = jnp.concatenate([pred_feat[:, 0], jnp.zeros((padr,), jnp.float32)])
    p1 = jnp.concatenate([pred_feat[:, 1], jnp.zeros((padr,), jnp.float32)])
    p2 = jnp.concatenate([pred_feat[:, 2], jnp.zeros((padr,), jnp.float32)])
    p3 = jnp.concatenate([pred_feat[:, 3], jnp.zeros((padr,), jnp.float32)])
    out = _tc_loss(po, p0, p1, p2, p3, m, gz, gy, gx, gf)
    return out.reshape(())
